# baseline (speedup 1.0000x reference)
import jax
import jax.numpy as jnp
from jax.experimental import pallas as pl
from jax.experimental.pallas import tpu as pltpu

_HEADS = 4
_DH = 16
_SCALE = 1.0 / (_DH ** 0.5)
_NB = 5


def _body(q_ref, k_ref, Ht_ref, ef_ref, packA_ref, packB_ref, out_ref,
          runmax, runsum, runP, k2s_s, v2t_s, hhT_s, hh2_s,
          bn1s, bn1q, bn2s, bn2q, sc1_s, sh1_s, sc2_s, sh2_s):
    f32 = jnp.float32
    oc = 64
    m = 64
    neg_inf = float("-inf")
    p = pl.program_id(0)
    i = pl.program_id(1)
    nb = pl.num_programs(1)
    bsz = q_ref.shape[0]

    we2i = packA_ref[0:16, :]
    kqv_w = packA_ref[16:144, :]
    q2_w = packA_ref[144:272, :]
    kmix = packA_ref[272:400, :]
    wk1 = kmix[:, :oc]
    wk2 = kmix[:oc, oc:]
    wv2 = kmix[oc:, oc:]
    oo = packA_ref[400:464, :]
    wo1 = oo[:, :oc]
    wo2 = oo[:, oc:]
    womix = packA_ref[464:528, :]
    wo_ = womix[:, :oc]
    g1c = womix[:, 64:65]
    b1c = womix[:, 65:66]
    g2c = womix[:, 66:67]
    b2c = womix[:, 67:68]
    bffn2c = womix[:, 68:69]
    f1 = packA_ref[528:656, :]
    wffn1 = jnp.concatenate([f1[:64, :], f1[64:, :]], axis=1)
    wffn2 = packB_ref[:, :oc]
    bffn1c = packB_ref[:, 64:65]

    hsel = jax.lax.broadcasted_iota(jnp.int32, (_HEADS, 1, oc), 2) // _DH
    hid = jax.lax.broadcasted_iota(jnp.int32, (_HEADS, 1, oc), 0)
    mh3 = (hsel == hid).astype(f32)
    hselr = jax.lax.broadcasted_iota(jnp.int32, (_HEADS, oc, 1), 1) // _DH
    hidr = jax.lax.broadcasted_iota(jnp.int32, (_HEADS, oc, 1), 0)
    mhr = (hselr == hidr).astype(f32)

    n_inv = 1.0 / (bsz * _NB)

    @pl.when(p == 0)
    def _phase0():
        maddT = jnp.where(Ht_ref[0].astype(f32) > 0.0, 0.0, neg_inf)

        @pl.when(i == 0)
        def _init():
            runmax[...] = jnp.full_like(runmax, -1e30)
            runsum[...] = jnp.zeros_like(runsum)
            runP[...] = jnp.zeros_like(runP)

        ef = ef_ref[...] @ we2i
        khw = (ef @ wk1) * _SCALE
        khw4 = khw[None, :, :] * mh3
        khw4p = jnp.concatenate([khw4, jnp.zeros_like(khw4)], axis=2)
        KQV = k_ref[...] @ kqv_w
        s = jax.lax.dot_general(khw4p.reshape(_HEADS * m, 2 * oc), KQV,
                                (((1,), (1,)), ((), ())))
        s3 = s.reshape(_HEADS, m, -1) + maddT[None, :, :]
        sflat = s3.reshape(_HEADS * m, -1)
        bm = jnp.max(sflat, axis=1, keepdims=True)
        newm = jnp.maximum(runmax[...], bm)
        c = jnp.exp(runmax[...] - newm)
        e = jnp.exp(sflat - newm)
        runsum[...] = runsum[...] * c + jnp.sum(e, axis=1, keepdims=True)
        runP[...] = runP[...] * c + jax.lax.dot_general(
            e, KQV, (((1,), (0,)), ((), ())))
        runmax[...] = newm

        @pl.when(i == nb - 1)
        def _fin():
            rinv = jnp.where(runsum[...] > 0.0, 1.0 / runsum[...], 0.0)
            Vpart = runP[...].reshape(_HEADS, m, 2 * oc)[:, :, oc:]
            he_upd = jnp.sum(Vpart * rinv.reshape(_HEADS, m, 1) * mh3,
                             axis=0)
            new_he = he_upd @ wo1
            K2 = new_he @ (wk2 * _SCALE)
            k2s_s[...] = (K2[None, :, :] * mh3).reshape(_HEADS * m, oc)
            V2T = jax.lax.dot_general(wv2, new_he,
                                      (((0,), (1,)), ((), ())))
            v2t_s[...] = jnp.concatenate(
                [V2T * mhr[h] for h in range(_HEADS)], axis=1)

    @pl.when(p == 1)
    def _phase1():
        maddT = jnp.where(Ht_ref[0].astype(f32) > 0.0, 0.0, neg_inf)

        @pl.when(i == 0)
        def _init():
            bn1s[...] = jnp.zeros_like(bn1s)
            bn1q[...] = jnp.zeros_like(bn1q)

        qb = q_ref[...]
        Q2 = qb @ q2_w[:, :oc]
        s2 = jax.lax.dot_general(k2s_s[...], Q2,
                                 (((1,), (1,)), ((), ())))
        s23 = s2.reshape(_HEADS, m, -1) + maddT[None, :, :]
        cmax = jnp.maximum(jnp.max(s23, axis=1, keepdims=True), -1e30)
        ex2 = jnp.exp(s23 - cmax)
        csum = jnp.sum(ex2, axis=1, keepdims=True)
        rinv2 = jnp.where(csum > 0.0, 1.0 / csum, 0.0)
        a2 = (ex2 * rinv2).reshape(_HEADS * m, -1)
        node_updT = jax.lax.dot_general(v2t_s[...], a2,
                                        (((1,), (0,)), ((), ())))
        node_msgT = jax.lax.dot_general(wo2, node_updT,
                                        (((0,), (0,)), ((), ())))
        resT = jax.lax.dot_general(q2_w[:, oc:], qb,
                                   (((0,), (1,)), ((), ())))
        hhT = jax.lax.dot_general(wo_, node_msgT,
                                  (((0,), (0,)), ((), ()))) + resT
        hhT_s[i] = hhT
        bn1s[...] = bn1s[...] + jnp.sum(hhT, axis=1, keepdims=True)
        bn1q[...] = bn1q[...] + jnp.sum(hhT * hhT, axis=1, keepdims=True)

    @pl.when(p == 2)
    def _phase2():
        @pl.when(i == 0)
        def _stats():
            mu = bn1s[...] * n_inv
            var = bn1q[...] * n_inv - mu * mu
            sc = g1c / jnp.sqrt(var + 1e-5)
            sc1_s[...] = sc
            sh1_s[...] = b1c - mu * sc
            bn2s[...] = jnp.zeros_like(bn2s)
            bn2q[...] = jnp.zeros_like(bn2q)

        hin = hhT_s[i] * sc1_s[...] + sh1_s[...]
        t = jax.lax.dot_general(wffn1, hin,
                                (((0,), (0,)), ((), ()))) + bffn1c
        t = 0.5 * t * (1.0 + jax.lax.erf(t * (2.0 ** -0.5)))
        hh2 = jax.lax.dot_general(wffn2, t,
                                  (((0,), (0,)), ((), ()))) + bffn2c
        hh2 = hh2 + hin
        hh2_s[i] = hh2
        bn2s[...] = bn2s[...] + jnp.sum(hh2, axis=1, keepdims=True)
        bn2q[...] = bn2q[...] + jnp.sum(hh2 * hh2, axis=1, keepdims=True)

    @pl.when(p == 3)
    def _phase3():
        @pl.when(i == 0)
        def _stats():
            mu = bn2s[...] * n_inv
            var = bn2q[...] * n_inv - mu * mu
            sc = g2c / jnp.sqrt(var + 1e-5)
            sc2_s[...] = sc
            sh2_s[...] = b2c - mu * sc

        o = hh2_s[i] * sc2_s[...] + sh2_s[...]
        out_ref[...] = o.T


def kernel(graph, q, k, v, edge_feat, H, W_e2i, W_n2h_q, W_n2h_k, W_n2h_v,
           W_n2h_o, W_h2n_q, W_h2n_k, W_h2n_v, W_h2n_o, W_o, W_ffn1, b_ffn1,
           W_ffn2, b_ffn2, W_res, bn1_g, bn1_b, bn2_g, bn2_b):
    num_nodes = q.shape[0]
    oc = W_n2h_q.shape[1]
    m = edge_feat.shape[0]
    f32 = jnp.float32
    bsz = num_nodes // _NB
    packA = jnp.concatenate([
        W_e2i,
        jnp.concatenate([W_n2h_q, W_n2h_v], axis=1),
        jnp.concatenate([W_h2n_q, W_res], axis=1),
        jnp.concatenate([W_n2h_k,
                         jnp.concatenate([W_h2n_k, W_h2n_v], axis=0)], axis=1),
        jnp.concatenate([W_n2h_o, W_h2n_o], axis=1),
        jnp.concatenate([W_o, bn1_g[:, None], bn1_b[:, None], bn2_g[:, None],
                         bn2_b[:, None], b_ffn2[:, None],
                         jnp.zeros((oc, 59), f32)], axis=1),
        jnp.concatenate([W_ffn1[:, :128], W_ffn1[:, 128:]], axis=0),
    ], axis=0)
    packB = jnp.concatenate([W_ffn2, b_ffn1[:, None],
                             jnp.zeros((4 * oc, 63), f32)], axis=1)
    Ht8 = H.T.astype(jnp.int8).reshape(m, _NB, bsz).swapaxes(0, 1)
    hm = _HEADS * m
    grid = (4, _NB)
    return pl.pallas_call(
        _body,
        grid=grid,
        in_specs=[
            pl.BlockSpec((bsz, 128), lambda p, i: (jnp.where(p == 1, i, 0), 0)),
            pl.BlockSpec((bsz, 128), lambda p, i: (jnp.where(p == 0, i, 0), 0)),
            pl.BlockSpec((1, m, bsz),
                         lambda p, i: (jnp.where(p <= 1, i, 0), 0, 0)),
            pl.BlockSpec((m, 16), lambda p, i: (0, 0)),
            pl.BlockSpec((656, 128), lambda p, i: (0, 0)),
            pl.BlockSpec((4 * oc, 128), lambda p, i: (0, 0)),
        ],
        out_specs=pl.BlockSpec((bsz, oc),
                               lambda p, i: (jnp.where(p == 3, i, 0), 0)),
        out_shape=jax.ShapeDtypeStruct((num_nodes, oc), f32),
        scratch_shapes=[
            pltpu.VMEM((hm, 1), f32),
            pltpu.VMEM((hm, 1), f32),
            pltpu.VMEM((hm, 128), f32),
            pltpu.VMEM((hm, oc), f32),
            pltpu.VMEM((oc, hm), f32),
            pltpu.VMEM((_NB, oc, bsz), f32),
            pltpu.VMEM((_NB, oc, bsz), f32),
            pltpu.VMEM((oc, 1), f32),
            pltpu.VMEM((oc, 1), f32),
            pltpu.VMEM((oc, 1), f32),
            pltpu.VMEM((oc, 1), f32),
            pltpu.VMEM((oc, 1), f32),
            pltpu.VMEM((oc, 1), f32),
            pltpu.VMEM((oc, 1), f32),
            pltpu.VMEM((oc, 1), f32),
        ],
    )(q, k, Ht8, edge_feat, packA, packB)


# speedup vs baseline: 1.1842x; 1.1842x over previous
import jax
import jax.numpy as jnp
from jax.experimental import pallas as pl

_HEADS = 4
_DH = 16
_SCALE = 1.0 / (_DH ** 0.5)


def _hgt_kernel(q_ref, k_ref, Ht_ref, ef_ref,
                we2i_ref, wq1_ref, wk1_ref, wv1_ref, wo1_ref,
                wq2_ref, wk2_ref, wv2_ref, wo2_ref,
                wo_ref, wffn1_ref, bffn1_ref, wffn2_ref, bffn2_ref,
                wres_ref, g1_ref, b1_ref, g2_ref, b2_ref,
                out_ref):
    f32 = jnp.float32
    q = q_ref[...]
    k = k_ref[...]
    oc = wq1_ref.shape[1]
    m = ef_ref.shape[0]
    neg_inf = float("-inf")

    maddT = jnp.where(Ht_ref[...] > 0, 0.0, neg_inf)

    hsel = jax.lax.broadcasted_iota(jnp.int32, (_HEADS, 1, oc), 2) // _DH
    hid = jax.lax.broadcasted_iota(jnp.int32, (_HEADS, 1, oc), 0)
    mh3 = (hsel == hid).astype(f32)
    hselr = jax.lax.broadcasted_iota(jnp.int32, (_HEADS, oc, 1), 1) // _DH
    hidr = jax.lax.broadcasted_iota(jnp.int32, (_HEADS, oc, 1), 0)
    mhr = (hselr == hidr).astype(f32)

    ef = ef_ref[...] @ we2i_ref[...]
    khw = (ef @ wk1_ref[...]) * _SCALE

    kqv_w = jnp.concatenate([wq1_ref[...], wv1_ref[...]], axis=1)
    KQV = k @ kqv_w
    khw4 = khw[None, :, :] * mh3
    khw4p = jnp.concatenate([khw4, jnp.zeros_like(khw4)], axis=2)
    s = jax.lax.dot_general(khw4p.reshape(_HEADS * m, 2 * oc), KQV,
                            (((1,), (1,)), ((), ())))
    s3 = s.reshape(_HEADS, m, -1) + maddT[None, :, :]
    rmax = jnp.maximum(jnp.max(s3, axis=2, keepdims=True), -1e30)
    ex3 = jnp.exp(s3 - rmax)
    rsum = jnp.sum(ex3, axis=2, keepdims=True)
    rinv = jnp.where(rsum > 0.0, 1.0 / rsum, 0.0)
    P = jax.lax.dot_general(ex3.reshape(_HEADS * m, -1), KQV,
                            (((1,), (0,)), ((), ())))
    Vpart = P.reshape(_HEADS, m, 2 * oc)[:, :, oc:]
    he_upd = jnp.sum(Vpart * rinv * mh3, axis=0)
    new_he = he_upd @ wo1_ref[...]

    Q2 = q @ wq2_ref[...]
    K2 = new_he @ (wk2_ref[...] * _SCALE)
    V2T = jax.lax.dot_general(wv2_ref[...], new_he,
                              (((0,), (1,)), ((), ())))
    K2stack = (K2[None, :, :] * mh3).reshape(_HEADS * m, oc)
    s2 = jax.lax.dot_general(K2stack, Q2,
                             (((1,), (1,)), ((), ())))
    s23 = s2.reshape(_HEADS, m, -1) + maddT[None, :, :]
    cmax = jnp.maximum(jnp.max(s23, axis=1, keepdims=True), -1e30)
    ex2 = jnp.exp(s23 - cmax)
    csum = jnp.sum(ex2, axis=1, keepdims=True)
    rinv2 = jnp.where(csum > 0.0, 1.0 / csum, 0.0)
    a2 = (ex2 * rinv2).reshape(_HEADS * m, -1)
    V2Tstack = jnp.concatenate([V2T * mhr[h] for h in range(_HEADS)],
                               axis=1)
    node_updT = jax.lax.dot_general(V2Tstack, a2,
                                    (((1,), (0,)), ((), ())))

    node_msgT = jax.lax.dot_general(wo2_ref[...], node_updT,
                                    (((0,), (0,)), ((), ())))
    resT = jax.lax.dot_general(wres_ref[...], q,
                               (((0,), (1,)), ((), ())))
    hhT = jax.lax.dot_general(wo_ref[...], node_msgT,
                              (((0,), (0,)), ((), ()))) + resT
    n_inv = 1.0 / hhT.shape[1]
    mu = jnp.sum(hhT, axis=1, keepdims=True) * n_inv
    msq = jnp.sum(hhT * hhT, axis=1, keepdims=True) * n_inv
    sc1 = g1_ref[...] / jnp.sqrt(msq - mu * mu + 1e-5)
    hhT = hhT * sc1 + (b1_ref[...] - mu * sc1)
    hT_in = hhT
    tT = jax.lax.dot_general(wffn1_ref[...], hhT,
                             (((0,), (0,)), ((), ()))) + bffn1_ref[...]
    tT = 0.5 * tT * (1.0 + jax.lax.erf(tT * (2.0 ** -0.5)))
    hhT = jax.lax.dot_general(wffn2_ref[...], tT,
                              (((0,), (0,)), ((), ()))) + bffn2_ref[...]
    hhT = hhT + hT_in
    mu = jnp.sum(hhT, axis=1, keepdims=True) * n_inv
    msq = jnp.sum(hhT * hhT, axis=1, keepdims=True) * n_inv
    sc2 = g2_ref[...] / jnp.sqrt(msq - mu * mu + 1e-5)
    outT = hhT * sc2 + (b2_ref[...] - mu * sc2)
    out_ref[...] = outT.T


def kernel(graph, q, k, v, edge_feat, H, W_e2i, W_n2h_q, W_n2h_k, W_n2h_v,
           W_n2h_o, W_h2n_q, W_h2n_k, W_h2n_v, W_h2n_o, W_o, W_ffn1, b_ffn1,
           W_ffn2, b_ffn2, W_res, bn1_g, bn1_b, bn2_g, bn2_b):
    num_nodes = q.shape[0]
    oc = W_n2h_q.shape[1]
    return pl.pallas_call(
        _hgt_kernel,
        out_shape=jax.ShapeDtypeStruct((num_nodes, oc), jnp.float32),
    )(q, k, H.T, edge_feat,
      W_e2i, W_n2h_q, W_n2h_k, W_n2h_v, W_n2h_o,
      W_h2n_q, W_h2n_k, W_h2n_v, W_h2n_o,
      W_o, W_ffn1, b_ffn1.reshape(-1, 1), W_ffn2, b_ffn2.reshape(-1, 1),
      W_res, bn1_g.reshape(-1, 1), bn1_b.reshape(-1, 1),
      bn2_g.reshape(-1, 1), bn2_b.reshape(-1, 1))
